# Initial kernel scaffold; baseline (speedup 1.0000x reference)
import jax
import jax.numpy as jnp
from jax.experimental import pallas as pl


def kernel(indices, table):
    raise NotImplementedError("write your pallas kernel here")


# speedup vs baseline: 1.2970x; 1.2970x over previous
import functools

import jax
import jax.numpy as jnp
from jax import lax
from jax.experimental import pallas as pl
from jax.experimental.pallas import tpu as pltpu
from jax.experimental.pallas import tpu_sc as plsc

_D = 32
_CHUNK = 128
_NC = 2
_NS = 16
_NW = _NC * _NS
_K = 8


def _gather_call(idx2d, table):
    n_chunks = idx2d.shape[0]
    chunks_w = n_chunks // _NW
    n_it = chunks_w // _K

    mesh = plsc.VectorSubcoreMesh(core_axis_name="c", subcore_axis_name="s")

    @functools.partial(
        pl.kernel,
        mesh=mesh,
        compiler_params=pltpu.CompilerParams(use_tc_tiling_on_sc=False),
        out_type=jax.ShapeDtypeStruct((n_chunks, _CHUNK, _D), jnp.float32),
        scratch_types=[
            pltpu.VMEM((chunks_w, _CHUNK), jnp.int32),
            pltpu.VMEM((_K, _CHUNK, _D), jnp.float32),
            pltpu.SemaphoreType.DMA,
        ],
    )
    def body(idx_hbm, table_hbm, out_hbm, idx_v, rows_v, sem):
        wid = lax.axis_index("s") * _NC + lax.axis_index("c")
        base = wid * chunks_w
        pltpu.sync_copy(idx_hbm.at[pl.ds(base, chunks_w)], idx_v)

        @pl.loop(0, n_it)
        def _step(g):
            copies = [
                pltpu.async_copy(
                    table_hbm.at[idx_v.at[g * _K + j]], rows_v.at[j], sem)
                for j in range(_K)
            ]
            for c in copies:
                c.wait()
            pltpu.sync_copy(rows_v, out_hbm.at[pl.ds(base + g * _K, _K)])

    return body(idx2d, table)


def kernel(indices, table):
    b, h = indices.shape
    idx2d = indices.astype(jnp.int32).reshape(-1, _CHUNK)
    out = _gather_call(idx2d, table)
    return out.reshape(b, h, _D)


# speedup vs baseline: 1.3109x; 1.0107x over previous
import functools

import jax
import jax.numpy as jnp
from jax import lax
from jax.experimental import pallas as pl
from jax.experimental.pallas import tpu as pltpu
from jax.experimental.pallas import tpu_sc as plsc

_D = 32
_CHUNK = 128
_NC = 2
_NS = 16
_NW = _NC * _NS
_K = 10
_STEP_BYTES = _K * _CHUNK * _D * 4


def _gather_call(idx2d, table):
    n_chunks = idx2d.shape[0]
    chunks_w = n_chunks // _NW
    n_it = chunks_w // _K

    mesh = plsc.VectorSubcoreMesh(core_axis_name="c", subcore_axis_name="s")

    @functools.partial(
        pl.kernel,
        mesh=mesh,
        compiler_params=pltpu.CompilerParams(use_tc_tiling_on_sc=False),
        out_type=jax.ShapeDtypeStruct((n_chunks, _CHUNK, _D), jnp.float32),
        scratch_types=[
            pltpu.VMEM((chunks_w, _CHUNK), jnp.int32),
            pltpu.VMEM((2, _K, _CHUNK, _D), jnp.float32),
            pltpu.SemaphoreType.DMA((2,)),
            pltpu.SemaphoreType.DMA((2,)),
        ],
    )
    def body(idx_hbm, table_hbm, out_hbm, idx_v, rows_v, gsem, ssem):
        wid = lax.axis_index("s") * _NC + lax.axis_index("c")
        base = wid * chunks_w
        pltpu.sync_copy(idx_hbm.at[pl.ds(base, chunks_w)], idx_v)

        def fire(g, b):
            for j in range(_K):
                pltpu.async_copy(table_hbm.at[idx_v.at[g * _K + j]],
                                 rows_v.at[b, j], gsem.at[b])

        def store(g, b):
            pltpu.async_copy(rows_v.at[b], out_hbm.at[pl.ds(base + g * _K, _K)],
                             ssem.at[b])

        def drain_gather(b):
            pltpu.make_async_copy(
                out_hbm.at[pl.ds(0, _K)], rows_v.at[b], gsem.at[b]).wait()

        def drain_store(b):
            pltpu.make_async_copy(
                rows_v.at[b], out_hbm.at[pl.ds(0, _K)], ssem.at[b]).wait()

        fire(0, 0)

        @pl.loop(0, n_it, step=2)
        def _step(g0):
            @pl.when(g0 > 0)
            def _():
                drain_store(1)
            fire(g0 + 1, 1)
            drain_gather(0)
            store(g0, 0)

            @pl.when(g0 + 2 < n_it)
            def _():
                drain_store(0)
                fire(g0 + 2, 0)
            drain_gather(1)
            store(g0 + 1, 1)

        drain_store(0)
        drain_store(1)

    return body(idx2d, table)


def kernel(indices, table):
    b, h = indices.shape
    idx2d = indices.astype(jnp.int32).reshape(-1, _CHUNK)
    out = _gather_call(idx2d, table)
    return out.reshape(b, h, _D)


# speedup vs baseline: 2.6037x; 1.9862x over previous
import functools

import jax
import jax.numpy as jnp
from jax import lax
from jax.experimental import pallas as pl
from jax.experimental.pallas import tpu as pltpu
from jax.experimental.pallas import tpu_sc as plsc

_D = 32
_CHUNK = 128
_NC = 2
_NS = 16
_NW = _NC * _NS
_K = 2
_GB = 4
_HB = 16384 // _CHUNK


def _gather_call(idx2d, table, n_hist):
    n_blocks = idx2d.shape[0]
    blocks_w = n_blocks // _NW
    n_it = blocks_w // _K

    mesh = plsc.VectorSubcoreMesh(core_axis_name="c", subcore_axis_name="s")

    @functools.partial(
        pl.kernel,
        mesh=mesh,
        compiler_params=pltpu.CompilerParams(
            use_tc_tiling_on_sc=False, needs_layout_passes=False),
        out_type=jax.ShapeDtypeStruct(
            (n_hist, _D // 8, _HB, 8, _CHUNK), jnp.float32),
        scratch_types=[
            pltpu.VMEM((blocks_w, _CHUNK), jnp.int32),
            pltpu.VMEM((_GB, _K, _CHUNK, _D), jnp.float32),
            pltpu.VMEM((2, _D // 8, _K, 8, _CHUNK), jnp.float32),
            pltpu.SemaphoreType.DMA((_GB,)),
            pltpu.SemaphoreType.DMA((2,)),
        ],
    )
    def body(idx_hbm, table_hbm, out_hbm, idx_v, grows, tbuf, gsem, ssem):
        wid = lax.axis_index("s") * _NC + lax.axis_index("c")
        base = wid * blocks_w
        pltpu.sync_copy(idx_hbm.at[pl.ds(base, blocks_w)], idx_v)
        iota16 = lax.iota(jnp.int32, 16)

        def fire(s, g):
            for j in range(_K):
                pltpu.async_copy(table_hbm.at[idx_v.at[s * _K + j]],
                                 grows.at[g, j], gsem.at[g])

        kvecs = [jnp.full((16,), k, jnp.int32) for k in range(_K)]
        ediags = [(e0 + iota16) % _D for e0 in range(_D)]

        def transpose_k(g, t, k):
            @pl.loop(0, _CHUNK // 16)
            def _g(i):
                b0 = i * 16
                bvec = iota16 + b0
                for e0 in range(_D):
                    ed = ediags[e0]
                    et = ed // 8
                    ei = ed % 8
                    v = plsc.load_gather(grows.at[g, k], [bvec, ed])
                    plsc.store_scatter(tbuf.at[t], [et, kvecs[k], ei, bvec], v)

        def store(s, t):
            g0 = base + s * _K
            h = g0 // _HB
            bt = g0 % _HB
            for e_t in range(_D // 8):
                pltpu.async_copy(tbuf.at[t, e_t],
                                 out_hbm.at[h, e_t, pl.ds(bt, _K)],
                                 ssem.at[t])

        def drain_transpose(g, t):
            for j in range(_K):
                pltpu.make_async_copy(
                    table_hbm.at[pl.ds(0, _CHUNK)], grows.at[g, j],
                    gsem.at[g]).wait()
                transpose_k(g, t, j)

        def drain_store(t):
            for e_t in range(_D // 8):
                pltpu.make_async_copy(
                    tbuf.at[t, e_t], out_hbm.at[0, e_t, pl.ds(0, _K)],
                    ssem.at[t]).wait()

        fire(0, 0)
        fire(1, 1)
        fire(2, 2)

        @pl.loop(0, n_it, step=4)
        def _step(s0):
            for j in range(4):
                s = s0 + j
                g = j
                t = j % 2

                @pl.when(s + 3 < n_it)
                def _():
                    fire(s + 3, (j + 3) % 4)

                if j < 2:
                    @pl.when(s0 > 0)
                    def _():
                        drain_store(t)
                else:
                    drain_store(t)
                drain_transpose(g, t)
                store(s, t)

        drain_store(0)
        drain_store(1)

    return body(idx2d, table)


def kernel(indices, table):
    b, h = indices.shape
    idx2d = indices.astype(jnp.int32).T.reshape(-1, _CHUNK)
    out5d = _gather_call(idx2d, table, h)
    return out5d.transpose((2, 4, 0, 1, 3)).reshape(b, h, _D)
